# baseline (speedup 1.0000x reference)
import functools

import jax
import jax.numpy as jnp
from jax import lax
from jax.experimental import pallas as pl
from jax.experimental.pallas import tpu as pltpu
from jax.experimental.pallas import tpu_sc as plsc

_CHUNK = 128
_NC = 2
_NS = 16
_NSLOT = 6


@functools.lru_cache(maxsize=None)
def _make_agg_kernel(npad: int, e_pad: int, cb: int, with_deg: bool):
  zrows = npad // _NS
  et = e_pad // (_NC * _NS)
  nch = et // _CHUNK
  assert nch % _NSLOT == 0
  eh = e_pad // _NC
  mesh = plsc.VectorSubcoreMesh(core_axis_name="c", subcore_axis_name="s",
                                num_cores=_NC, num_subcores=_NS)
  n_out = cb + (1 if with_deg else 0)

  @functools.partial(
      pl.kernel,
      out_type=jax.ShapeDtypeStruct((_NC, n_out, npad, 32), jnp.float32),
      mesh=mesh,
      scratch_types=[
          [pltpu.VMEM((2, _CHUNK), jnp.int32) for _ in range(_NSLOT)],
          [pltpu.VMEM((_CHUNK, 32), jnp.float32) for _ in range(_NSLOT)],
          pltpu.VMEM_SHARED((npad, 32), jnp.float32),
          [pltpu.SemaphoreType.DMA for _ in range(_NSLOT)],
          [pltpu.SemaphoreType.DMA for _ in range(_NSLOT)],
          [pltpu.SemaphoreType.DMA for _ in range(_NSLOT)],
      ],
      compiler_params=pltpu.CompilerParams(use_tc_tiling_on_sc=False),
  )
  def agg_kernel(eidx_hbm, zeros_hbm, ones_hbm, tabs_hbm, out_hbm,
                 ev, rv, acc, si, sg, ss):
    c = lax.axis_index("c")
    s = lax.axis_index("s")
    r0 = pl.multiple_of(s * zrows, 8)
    base = c * eh + s * et

    def idx_start(j, g):
      off = pl.multiple_of(base + g * _CHUNK, 8)
      pltpu.async_copy(eidx_hbm.at[:, pl.ds(off, _CHUNK)], ev[j], si[j])

    def idx_wait(j):
      pltpu.make_async_copy(eidx_hbm.at[:, pl.ds(0, _CHUNK)], ev[j],
                            si[j]).wait()

    def zero_acc():
      pltpu.sync_copy(zeros_hbm, acc.at[pl.ds(r0, zrows)])
      plsc.subcore_barrier()

    def copy_out(b):
      plsc.subcore_barrier()
      pltpu.sync_copy(acc.at[pl.ds(r0, zrows)],
                      out_hbm.at[c, b, pl.ds(r0, zrows)])
      plsc.subcore_barrier()

    def prologue():
      for j in range(_NSLOT):
        idx_start(j, j)

    if with_deg:
      zero_acc()
      ones_v = rv[0]
      pltpu.sync_copy(ones_hbm, ones_v)
      prologue()

      def deg_body(q, carry):
        g0 = _NSLOT * q
        for j in range(_NSLOT):
          idx_wait(j)
          pltpu.async_copy(ones_v, acc.at[ev[j].at[1]], ss[j], add=True)
        for j in range(_NSLOT):
          pltpu.make_async_copy(ones_v, acc.at[ev[j].at[1]], ss[j]).wait()

          @pl.when(g0 + _NSLOT + j < nch)
          def _():
            idx_start(j, g0 + _NSLOT + j)

        return carry

      lax.fori_loop(0, nch // _NSLOT, deg_body, 0)
      copy_out(cb)

    for b in range(cb):
      tab = tabs_hbm.at[b]
      zero_acc()
      prologue()

      def body(q, carry, tab=tab):
        g0 = _NSLOT * q
        for j in range(_NSLOT):
          idx_wait(j)
          pltpu.async_copy(tab.at[ev[j].at[0]], rv[j], sg[j])
        for j in range(_NSLOT):
          pltpu.make_async_copy(tab.at[ev[j].at[0]], rv[j], sg[j]).wait()
          pltpu.async_copy(rv[j], acc.at[ev[j].at[1]], ss[j], add=True)
        for j in range(_NSLOT):
          pltpu.make_async_copy(rv[j], acc.at[ev[j].at[1]], ss[j]).wait()

          @pl.when(g0 + _NSLOT + j < nch)
          def _():
            idx_start(j, g0 + _NSLOT + j)

        return carry

      lax.fori_loop(0, nch // _NSLOT, body, 0)
      copy_out(b)

  return agg_kernel


_R = 1000


def _full(shape):
  return pl.BlockSpec(shape, lambda i: (0,) * len(shape))


def _rows(shape):
  return pl.BlockSpec(shape, lambda i: (i,) + (0,) * (len(shape) - 1))


def _mid(shape):
  return pl.BlockSpec(shape, lambda i: (0, i, 0))


def _adapter_body(x_ref, wa1, ba1, wa2, ba2, g, b, out_ref):
  t = jnp.maximum(jnp.dot(x_ref[...], wa1[...],
                          preferred_element_type=jnp.float32) + ba1[...], 0.0)
  h = jnp.dot(t, wa2[...], preferred_element_type=jnp.float32) + ba2[...]
  mu = jnp.mean(h, axis=-1, keepdims=True)
  var = jnp.mean((h - mu) * (h - mu), axis=-1, keepdims=True)
  y = (h - mu) * lax.rsqrt(var + 1e-5) * g[...] + b[...]
  for k in range(4):
    out_ref[k] = y[:, 32 * k:32 * k + 32]


def _deg_of(ref, blk):
  d = ref[0, blk, :, 0:1] + ref[1, blk, :, 0:1]
  return jnp.maximum(d, 1.0)


def _cat_agg(ref, nblk):
  return jnp.concatenate([ref[0, k] + ref[1, k] for k in range(nblk)], axis=-1)


def _layer1_body(h1t_ref, a1_ref, wl1, bl1, wr1, wl2, h2t_ref, p2t_ref):
  dg = _deg_of(a1_ref, 4)
  h1 = jnp.concatenate([h1t_ref[k] for k in range(4)], axis=-1)
  a1 = _cat_agg(a1_ref, 4)
  z = (jnp.dot(a1, wl1[...], preferred_element_type=jnp.float32) / dg
       + bl1[...]
       + jnp.dot(h1, wr1[...], preferred_element_type=jnp.float32))
  h2 = jnp.maximum(z, 0.0)
  p2 = jnp.dot(h2, wl2[...], preferred_element_type=jnp.float32)
  for k in range(8):
    h2t_ref[k] = h2[:, 32 * k:32 * k + 32]
  for k in range(4):
    p2t_ref[k] = p2[:, 32 * k:32 * k + 32]


def _layer2_body(h2t_ref, deg_ref, a2_ref, bl2, wr2, wl3, h3t_ref, p3t_ref):
  dg = _deg_of(deg_ref, 0)
  h2 = jnp.concatenate([h2t_ref[k] for k in range(8)], axis=-1)
  a2 = _cat_agg(a2_ref, 4)
  z = a2 / dg + bl2[...] + jnp.dot(h2, wr2[...],
                                   preferred_element_type=jnp.float32)
  h3 = jnp.maximum(z, 0.0)
  p3 = jnp.dot(h3, wl3[...], preferred_element_type=jnp.float32)
  for k in range(4):
    h3t_ref[k] = h3[:, 32 * k:32 * k + 32]
  for k in range(2):
    p3t_ref[k] = p3[:, 32 * k:32 * k + 32]


def _layer3_body(h3t_ref, deg_ref, a3_ref, bl3, wr3, w_out, b_out, out_ref):
  dg = _deg_of(deg_ref, 0)
  h3 = jnp.concatenate([h3t_ref[k] for k in range(4)], axis=-1)
  a3 = _cat_agg(a3_ref, 2)
  z = a3 / dg + bl3[...] + jnp.dot(h3, wr3[...],
                                   preferred_element_type=jnp.float32)
  h4 = jnp.maximum(z, 0.0)
  out_ref[...] = (jnp.dot(h4, w_out[...], preferred_element_type=jnp.float32)
                  + b_out[...])


def kernel(x, edge_index, input_cards, W_a1, b_a1, W_a2, b_a2, ln_g, ln_b,
           Wl1, bl1, Wr1, Wl2, bl2, Wr2, Wl3, bl3, Wr3, W_out, b_out):
  n = x.shape[0]
  e = edge_index.shape[1]
  npad = ((n + 1 + 127) // 128) * 128
  per = _NC * _NS * _CHUNK * _NSLOT
  e_pad = ((e + per - 1) // per) * per
  zrows = npad // _NS
  grid = (n // _R,)
  assert n % _R == 0

  eidx = edge_index
  if e_pad > e:
    pad = e_pad - e
    eidx = jnp.concatenate(
        [eidx, jnp.stack([jnp.zeros((pad,), jnp.int32),
                          jnp.full((pad,), n, jnp.int32)])], axis=1)
  x7 = jnp.concatenate([x, input_cards[:, None].astype(jnp.float32)], axis=1)
  zeros32 = jnp.zeros((zrows, 32), jnp.float32)
  ones32 = jnp.ones((_CHUNK, 32), jnp.float32)
  b_a1r = b_a1[None, :]
  b_a2r = b_a2[None, :]
  ln_gr = ln_g[None, :]
  ln_br = ln_b[None, :]
  bl1r = bl1[None, :]
  bl2r = bl2[None, :]
  bl3r = bl3[None, :]
  b_outr = b_out[None, :]

  def deg_spec():
    return pl.BlockSpec((2, 1, _R, 32), lambda i: (0, 4, i, 0))

  h1t = pl.pallas_call(
      _adapter_body,
      grid=grid,
      in_specs=[_rows((_R, 7)), _full((7, 64)), _full((1, 64)),
                _full((64, 128)), _full((1, 128)),
                _full((1, 128)), _full((1, 128))],
      out_specs=_mid((4, _R, 32)),
      out_shape=jax.ShapeDtypeStruct((4, n, 32), jnp.float32),
  )(x7, W_a1, b_a1r, W_a2, b_a2r, ln_gr, ln_br)

  a1 = _make_agg_kernel(npad, e_pad, 4, True)(eidx, zeros32, ones32, h1t)

  h2t, p2t = pl.pallas_call(
      _layer1_body,
      grid=grid,
      in_specs=[_mid((4, _R, 32)),
                pl.BlockSpec((2, 5, _R, 32), lambda i: (0, 0, i, 0)),
                _full((128, 256)), _full((1, 256)), _full((128, 256)),
                _full((256, 128))],
      out_specs=[_mid((8, _R, 32)), _mid((4, _R, 32))],
      out_shape=[jax.ShapeDtypeStruct((8, n, 32), jnp.float32),
                 jax.ShapeDtypeStruct((4, n, 32), jnp.float32)],
  )(h1t, a1, Wl1, bl1r, Wr1, Wl2)

  a2 = _make_agg_kernel(npad, e_pad, 4, False)(eidx, zeros32, ones32, p2t)

  h3t, p3t = pl.pallas_call(
      _layer2_body,
      grid=grid,
      in_specs=[_mid((8, _R, 32)), deg_spec(),
                pl.BlockSpec((2, 4, _R, 32), lambda i: (0, 0, i, 0)),
                _full((1, 128)), _full((256, 128)), _full((128, 64))],
      out_specs=[_mid((4, _R, 32)), _mid((2, _R, 32))],
      out_shape=[jax.ShapeDtypeStruct((4, n, 32), jnp.float32),
                 jax.ShapeDtypeStruct((2, n, 32), jnp.float32)],
  )(h2t, a1, a2, bl2r, Wr2, Wl3)

  a3 = _make_agg_kernel(npad, e_pad, 2, False)(eidx, zeros32, ones32, p3t)

  logits = pl.pallas_call(
      _layer3_body,
      grid=grid,
      in_specs=[_mid((4, _R, 32)), deg_spec(),
                pl.BlockSpec((2, 2, _R, 32), lambda i: (0, 0, i, 0)),
                _full((1, 64)), _full((128, 64)),
                _full((64, 110)), _full((1, 110))],
      out_specs=_rows((_R, 110)),
      out_shape=jax.ShapeDtypeStruct((n, 110), jnp.float32),
  )(h3t, a1, a3, bl3r, Wr3, W_out, b_outr)

  return logits


# speedup vs baseline: 1.3388x; 1.3388x over previous
import functools

import jax
import jax.numpy as jnp
from jax import lax
from jax.experimental import pallas as pl
from jax.experimental.pallas import tpu as pltpu
from jax.experimental.pallas import tpu_sc as plsc

_CHUNK = 128
_NC = 2
_NS = 16
_NSLOT = 4


@functools.lru_cache(maxsize=None)
def _make_agg_kernel(npad: int, e_pad: int, cb: int, with_deg: bool):
  zrows = npad // _NS
  et = e_pad // (_NC * _NS)
  nch = et // _CHUNK
  assert nch % _NSLOT == 0
  eh = e_pad // _NC
  mesh = plsc.VectorSubcoreMesh(core_axis_name="c", subcore_axis_name="s",
                                num_cores=_NC, num_subcores=_NS)
  n_out = cb + (1 if with_deg else 0)

  @functools.partial(
      pl.kernel,
      out_type=jax.ShapeDtypeStruct((_NC, n_out, npad, 32), jnp.float32),
      mesh=mesh,
      scratch_types=[
          [pltpu.VMEM((2, _CHUNK), jnp.int32) for _ in range(_NSLOT)],
          [pltpu.VMEM((_CHUNK, 32), jnp.float32) for _ in range(_NSLOT)],
          pltpu.VMEM_SHARED((npad, 32), jnp.float32),
          [pltpu.SemaphoreType.DMA for _ in range(_NSLOT)],
          [pltpu.SemaphoreType.DMA for _ in range(_NSLOT)],
          [pltpu.SemaphoreType.DMA for _ in range(_NSLOT)],
      ],
      compiler_params=pltpu.CompilerParams(use_tc_tiling_on_sc=False),
  )
  def agg_kernel(eidx_hbm, zeros_hbm, ones_hbm, tabs_hbm, out_hbm,
                 ev, rv, acc, si, sg, ss):
    c = lax.axis_index("c")
    s = lax.axis_index("s")
    r0 = pl.multiple_of(s * zrows, 8)
    base = c * eh + s * et

    def idx_start(j, g):
      off = pl.multiple_of(base + g * _CHUNK, 8)
      pltpu.async_copy(eidx_hbm.at[:, pl.ds(off, _CHUNK)], ev[j], si[j])

    def idx_wait(j):
      pltpu.make_async_copy(eidx_hbm.at[:, pl.ds(0, _CHUNK)], ev[j],
                            si[j]).wait()

    def zero_acc():
      pltpu.sync_copy(zeros_hbm, acc.at[pl.ds(r0, zrows)])
      plsc.subcore_barrier()

    def copy_out(b):
      plsc.subcore_barrier()
      pltpu.sync_copy(acc.at[pl.ds(r0, zrows)],
                      out_hbm.at[c, b, pl.ds(r0, zrows)])
      plsc.subcore_barrier()

    def prologue():
      for j in range(_NSLOT):
        idx_start(j, j)

    if with_deg:
      zero_acc()
      ones_v = rv[0]
      pltpu.sync_copy(ones_hbm, ones_v)
      prologue()

      def deg_body(q, carry):
        g0 = _NSLOT * q
        for j in range(_NSLOT):
          idx_wait(j)
          pltpu.async_copy(ones_v, acc.at[ev[j].at[1]], ss[j], add=True)
        for j in range(_NSLOT):
          pltpu.make_async_copy(ones_v, acc.at[ev[j].at[1]], ss[j]).wait()

          @pl.when(g0 + _NSLOT + j < nch)
          def _():
            idx_start(j, g0 + _NSLOT + j)

        return carry

      lax.fori_loop(0, nch // _NSLOT, deg_body, 0)
      copy_out(cb)

    for b in range(cb):
      tab = tabs_hbm.at[b]
      zero_acc()
      prologue()

      def body(q, carry, tab=tab):
        g0 = _NSLOT * q
        for j in range(_NSLOT):
          idx_wait(j)
          pltpu.async_copy(tab.at[ev[j].at[0]], rv[j], sg[j])
        for j in range(_NSLOT):
          pltpu.make_async_copy(tab.at[ev[j].at[0]], rv[j], sg[j]).wait()
          pltpu.async_copy(rv[j], acc.at[ev[j].at[1]], ss[j], add=True)
        for j in range(_NSLOT):
          pltpu.make_async_copy(rv[j], acc.at[ev[j].at[1]], ss[j]).wait()

          @pl.when(g0 + _NSLOT + j < nch)
          def _():
            idx_start(j, g0 + _NSLOT + j)

        return carry

      lax.fori_loop(0, nch // _NSLOT, body, 0)
      copy_out(b)

  return agg_kernel


_R = 1000


def _full(shape):
  return pl.BlockSpec(shape, lambda i: (0,) * len(shape))


def _rows(shape):
  return pl.BlockSpec(shape, lambda i: (i,) + (0,) * (len(shape) - 1))


def _mid(shape):
  return pl.BlockSpec(shape, lambda i: (0, i, 0))


def _adapter_body(x_ref, wa1, ba1, wa2, ba2, g, b, out_ref):
  t = jnp.maximum(jnp.dot(x_ref[...], wa1[...],
                          preferred_element_type=jnp.float32) + ba1[...], 0.0)
  h = jnp.dot(t, wa2[...], preferred_element_type=jnp.float32) + ba2[...]
  mu = jnp.mean(h, axis=-1, keepdims=True)
  var = jnp.mean((h - mu) * (h - mu), axis=-1, keepdims=True)
  y = (h - mu) * lax.rsqrt(var + 1e-5) * g[...] + b[...]
  for k in range(4):
    out_ref[k] = y[:, 32 * k:32 * k + 32]


def _deg_of(ref, blk):
  d = ref[0, blk, :, 0:1] + ref[1, blk, :, 0:1]
  return jnp.maximum(d, 1.0)


def _cat_agg(ref, nblk):
  return jnp.concatenate([ref[0, k] + ref[1, k] for k in range(nblk)], axis=-1)


def _layer1_body(h1t_ref, a1_ref, wl1, bl1, wr1, wl2, h2t_ref, p2t_ref):
  dg = _deg_of(a1_ref, 4)
  h1 = jnp.concatenate([h1t_ref[k] for k in range(4)], axis=-1)
  a1 = _cat_agg(a1_ref, 4)
  z = (jnp.dot(a1, wl1[...], preferred_element_type=jnp.float32) / dg
       + bl1[...]
       + jnp.dot(h1, wr1[...], preferred_element_type=jnp.float32))
  h2 = jnp.maximum(z, 0.0)
  p2 = jnp.dot(h2, wl2[...], preferred_element_type=jnp.float32)
  for k in range(8):
    h2t_ref[k] = h2[:, 32 * k:32 * k + 32]
  for k in range(4):
    p2t_ref[k] = p2[:, 32 * k:32 * k + 32]


def _layer2_body(h2t_ref, deg_ref, a2_ref, bl2, wr2, wl3, h3t_ref, p3t_ref):
  dg = _deg_of(deg_ref, 0)
  h2 = jnp.concatenate([h2t_ref[k] for k in range(8)], axis=-1)
  a2 = _cat_agg(a2_ref, 4)
  z = a2 / dg + bl2[...] + jnp.dot(h2, wr2[...],
                                   preferred_element_type=jnp.float32)
  h3 = jnp.maximum(z, 0.0)
  p3 = jnp.dot(h3, wl3[...], preferred_element_type=jnp.float32)
  for k in range(4):
    h3t_ref[k] = h3[:, 32 * k:32 * k + 32]
  for k in range(2):
    p3t_ref[k] = p3[:, 32 * k:32 * k + 32]


def _layer3_body(h3t_ref, deg_ref, a3_ref, bl3, wr3, w_out, b_out, out_ref):
  dg = _deg_of(deg_ref, 0)
  h3 = jnp.concatenate([h3t_ref[k] for k in range(4)], axis=-1)
  a3 = _cat_agg(a3_ref, 2)
  z = a3 / dg + bl3[...] + jnp.dot(h3, wr3[...],
                                   preferred_element_type=jnp.float32)
  h4 = jnp.maximum(z, 0.0)
  out_ref[...] = (jnp.dot(h4, w_out[...], preferred_element_type=jnp.float32)
                  + b_out[...])


def kernel(x, edge_index, input_cards, W_a1, b_a1, W_a2, b_a2, ln_g, ln_b,
           Wl1, bl1, Wr1, Wl2, bl2, Wr2, Wl3, bl3, Wr3, W_out, b_out):
  n = x.shape[0]
  e = edge_index.shape[1]
  npad = ((n + 1 + 127) // 128) * 128
  per = _NC * _NS * _CHUNK * _NSLOT
  e_pad = ((e + per - 1) // per) * per
  zrows = npad // _NS
  grid = (n // _R,)
  assert n % _R == 0

  eidx = edge_index
  if e_pad > e:
    pad = e_pad - e
    eidx = jnp.concatenate(
        [eidx, jnp.stack([jnp.zeros((pad,), jnp.int32),
                          jnp.full((pad,), n, jnp.int32)])], axis=1)
  x7 = jnp.concatenate([x, input_cards[:, None].astype(jnp.float32)], axis=1)
  zeros32 = jnp.zeros((zrows, 32), jnp.float32)
  ones32 = jnp.ones((_CHUNK, 32), jnp.float32)
  b_a1r = b_a1[None, :]
  b_a2r = b_a2[None, :]
  ln_gr = ln_g[None, :]
  ln_br = ln_b[None, :]
  bl1r = bl1[None, :]
  bl2r = bl2[None, :]
  bl3r = bl3[None, :]
  b_outr = b_out[None, :]

  def deg_spec():
    return pl.BlockSpec((2, 1, _R, 32), lambda i: (0, 4, i, 0))

  h1t = pl.pallas_call(
      _adapter_body,
      grid=grid,
      in_specs=[_rows((_R, 7)), _full((7, 64)), _full((1, 64)),
                _full((64, 128)), _full((1, 128)),
                _full((1, 128)), _full((1, 128))],
      out_specs=_mid((4, _R, 32)),
      out_shape=jax.ShapeDtypeStruct((4, n, 32), jnp.float32),
  )(x7, W_a1, b_a1r, W_a2, b_a2r, ln_gr, ln_br)

  a1 = _make_agg_kernel(npad, e_pad, 4, True)(eidx, zeros32, ones32, h1t)

  h2t, p2t = pl.pallas_call(
      _layer1_body,
      grid=grid,
      in_specs=[_mid((4, _R, 32)),
                pl.BlockSpec((2, 5, _R, 32), lambda i: (0, 0, i, 0)),
                _full((128, 256)), _full((1, 256)), _full((128, 256)),
                _full((256, 128))],
      out_specs=[_mid((8, _R, 32)), _mid((4, _R, 32))],
      out_shape=[jax.ShapeDtypeStruct((8, n, 32), jnp.float32),
                 jax.ShapeDtypeStruct((4, n, 32), jnp.float32)],
  )(h1t, a1, Wl1, bl1r, Wr1, Wl2)

  a2 = _make_agg_kernel(npad, e_pad, 4, False)(eidx, zeros32, ones32, p2t)

  h3t, p3t = pl.pallas_call(
      _layer2_body,
      grid=grid,
      in_specs=[_mid((8, _R, 32)), deg_spec(),
                pl.BlockSpec((2, 4, _R, 32), lambda i: (0, 0, i, 0)),
                _full((1, 128)), _full((256, 128)), _full((128, 64))],
      out_specs=[_mid((4, _R, 32)), _mid((2, _R, 32))],
      out_shape=[jax.ShapeDtypeStruct((4, n, 32), jnp.float32),
                 jax.ShapeDtypeStruct((2, n, 32), jnp.float32)],
  )(h2t, a1, a2, bl2r, Wr2, Wl3)

  a3 = _make_agg_kernel(npad, e_pad, 2, False)(eidx, zeros32, ones32, p3t)

  logits = pl.pallas_call(
      _layer3_body,
      grid=grid,
      in_specs=[_mid((4, _R, 32)), deg_spec(),
                pl.BlockSpec((2, 2, _R, 32), lambda i: (0, 0, i, 0)),
                _full((1, 64)), _full((128, 64)),
                _full((64, 110)), _full((1, 110))],
      out_specs=_rows((_R, 110)),
      out_shape=jax.ShapeDtypeStruct((n, 110), jnp.float32),
  )(h3t, a1, a3, bl3r, Wr3, W_out, b_outr)

  return logits
